# baseline (speedup 1.0000x reference)
import functools

import jax
import jax.numpy as jnp
from jax import lax
from jax.experimental import pallas as pl
from jax.experimental.pallas import tpu as pltpu
from jax.experimental.pallas import tpu_sc as plsc

N = 10000
E = 160000
B = 100
DIM = 32
STEPS = 12
NCORE = 2
NSUB = 16
NW = NCORE * NSUB
EPW = E // NW
CH = 40
NCH = EPW // CH
MLD = 1000
NLD = EPW // MLD
SPL = MLD // CH
NPT = N // NSUB
BE = 640
E4 = E // 4
RB = 2000
S_PAD = 512
S_EPW = S_PAD // NW


def _leaky(v):
    return jax.nn.leaky_relu(v, 0.01)


def _lin0_body(x_ref, w_ref, b_ref, o_ref):
    o_ref[...] = _leaky(jnp.dot(x_ref[...], w_ref[...],
                                preferred_element_type=jnp.float32) + b_ref[...])


def _prep_w_body(ea4_ref, w1x_ref, b1x_ref, w2x_ref, b2x_ref, w_ref):
    he4 = _leaky(jnp.dot(ea4_ref[...], w1x_ref[...],
                         preferred_element_type=jnp.float32) + b1x_ref[...])
    he4b = he4.astype(jnp.bfloat16)
    for i in range(DIM):
        wi = jnp.dot(he4b, w2x_ref[i],
                     preferred_element_type=jnp.float32) + b2x_ref[i]
        w_ref[i] = wi.astype(jnp.bfloat16)


def _einsum_body(xj_ref, w_ref, ei_ref, msg_ref):
    xb = xj_ref[...].astype(jnp.bfloat16)
    acc = jnp.zeros((RB, 128), jnp.float32)
    for i in range(DIM):
        xe = jnp.dot(xb, ei_ref[i], preferred_element_type=jnp.float32)
        acc = acc + xe * w_ref[i].astype(jnp.float32)
    msg_ref[...] = acc


def _gru_body(agg_ref, out_ref, dinv_ref, cr_ref, cb_ref, wih_ref, bih_ref,
              whh_ref, bhh_ref, newo_ref):
    out = out_ref[...]
    agg = (agg_ref[0] + agg_ref[1]) * dinv_ref[...]
    m = _leaky(agg + jnp.dot(out, cr_ref[...],
                             preferred_element_type=jnp.float32) + cb_ref[...])
    gi = jnp.dot(m, wih_ref[...], preferred_element_type=jnp.float32) + bih_ref[...]
    gh = jnp.dot(out, whh_ref[...], preferred_element_type=jnp.float32) + bhh_ref[...]
    r = jax.nn.sigmoid(gi[:, 0:DIM] + gh[:, 0:DIM])
    z = jax.nn.sigmoid(gi[:, DIM:2 * DIM] + gh[:, DIM:2 * DIM])
    n = jnp.tanh(gi[:, 2 * DIM:] + r * gh[:, 2 * DIM:])
    newo = (1.0 - z) * n + z * out
    newo_ref[...] = newo


def _stem_body(rows_ref, w1_ref, b1_ref, w2_ref, b2_ref, o_ref):
    t = _leaky(jnp.dot(rows_ref[...], w1_ref[...],
                       preferred_element_type=jnp.float32) + b1_ref[...])
    o_ref[...] = jnp.dot(t, w2_ref[...],
                         preferred_element_type=jnp.float32) + b2_ref[...]


def _s2s_body(out_ref, bt_ref, wih_ref, bih_ref, whh_ref, bhh_ref,
              w3_ref, b3_ref, sout_ref):
    outT = jnp.transpose(out_ref[...])
    seg = jax.lax.broadcasted_iota(jnp.int32, (B, N), 0) == bt_ref[...]
    segf = seg.astype(jnp.float32)
    q_star = jnp.zeros((B, 2 * DIM), jnp.float32)
    hL = jnp.zeros((B, DIM), jnp.float32)
    cL = jnp.zeros((B, DIM), jnp.float32)
    for _ in range(3):
        gates = (jnp.dot(q_star, wih_ref[...], preferred_element_type=jnp.float32)
                 + bih_ref[...]
                 + jnp.dot(hL, whh_ref[...], preferred_element_type=jnp.float32)
                 + bhh_ref[...])
        ii = gates[:, 0:DIM]
        ff = gates[:, DIM:2 * DIM]
        gg = gates[:, 2 * DIM:3 * DIM]
        oo = gates[:, 3 * DIM:]
        cL = jax.nn.sigmoid(ff) * cL + jax.nn.sigmoid(ii) * jnp.tanh(gg)
        hL = jax.nn.sigmoid(oo) * jnp.tanh(cL)
        qbT = lax.dot_general(jnp.transpose(hL), segf, (((1,), (0,)), ((), ())),
                              preferred_element_type=jnp.float32)
        eT = jnp.sum(outT * qbT, axis=0, keepdims=True)
        em = jnp.where(seg, eT, -jnp.inf)
        mx = jnp.max(em, axis=1, keepdims=True)
        mx = jnp.where(jnp.isfinite(mx), mx, 0.0)
        p = jnp.where(seg, jnp.exp(eT - mx), 0.0)
        s = jnp.sum(p, axis=1, keepdims=True)
        s = jnp.where(s == 0.0, 1.0, s)
        a = p / s
        rvec = lax.dot_general(a, outT, (((1,), (1,)), ((), ())),
                               preferred_element_type=jnp.float32)
        q_star = jnp.concatenate([hL, rvec], axis=1)
    sout_ref[...] = jnp.dot(q_star, w3_ref[...],
                            preferred_element_type=jnp.float32) + b3_ref[...]


def _sc_gather_body(epw, mld, ch, table_hbm, idx_hbm, out_hbm, idx_v, rows_v, sem):
    wid = lax.axis_index("c") * NSUB + lax.axis_index("s")
    nld = epw // mld
    spl = mld // ch
    pltpu.sync_copy(idx_hbm.at[wid], idx_v)

    def round_(g, c):
        def fire(k, c2):
            pltpu.async_copy(table_hbm.at[idx_v.at[g * spl + k]],
                             rows_v.at[pl.ds(k * ch, ch)], sem)
            return c2

        lax.fori_loop(0, spl, fire, 0)

        def drain(k, c2):
            pltpu.make_async_copy(table_hbm.at[idx_v.at[0]],
                                  rows_v.at[pl.ds(0, ch)], sem).wait()
            return c2

        lax.fori_loop(0, spl, drain, 0)
        pltpu.sync_copy(rows_v, out_hbm.at[pl.ds(wid * epw + g * mld, mld)])
        return c

    lax.fori_loop(0, nld, round_, 0)


def _sc_scatter_body(msg_hbm, idx_hbm, zero_hbm, agg_hbm, idx_v, msg_v, agg_sh, sem):
    cid = lax.axis_index("c")
    sid = lax.axis_index("s")
    wid = cid * NSUB + sid
    pltpu.sync_copy(idx_hbm.at[wid], idx_v)
    pltpu.sync_copy(zero_hbm, agg_sh.at[pl.ds(sid * NPT, NPT)])
    plsc.subcore_barrier()

    def body(g, c):
        pltpu.sync_copy(msg_hbm.at[pl.ds(wid * EPW + g * MLD, MLD)], msg_v)

        def sub(k, c2):
            pltpu.sync_copy(msg_v.at[pl.ds(k * CH, CH)],
                            agg_sh.at[idx_v.at[g * SPL + k]], add=True)
            return c2

        lax.fori_loop(0, SPL, sub, 0)
        return c

    lax.fori_loop(0, NLD, body, 0)
    plsc.subcore_barrier()
    pltpu.sync_copy(agg_sh.at[pl.ds(sid * NPT, NPT)],
                    agg_hbm.at[cid, pl.ds(sid * NPT, NPT)])


_MESH = plsc.VectorSubcoreMesh(core_axis_name="c", subcore_axis_name="s")


def _sc_gather(table, idx, rows_out_shape, epw, mld, ch):
    dt = table.dtype
    nch = epw // ch
    return pl.kernel(
        functools.partial(_sc_gather_body, epw, mld, ch),
        out_type=jax.ShapeDtypeStruct(rows_out_shape, dt),
        mesh=_MESH,
        compiler_params=pltpu.CompilerParams(use_tc_tiling_on_sc=False),
        scratch_types=[
            pltpu.VMEM((nch, ch), jnp.int32),
            pltpu.VMEM((mld, DIM), dt),
            pltpu.SemaphoreType.DMA,
        ],
    )(table, idx)


def _sc_scatter(msg, idx, zero):
    return pl.kernel(
        _sc_scatter_body,
        out_type=jax.ShapeDtypeStruct((NCORE, N, DIM), jnp.float32),
        mesh=_MESH,
        compiler_params=pltpu.CompilerParams(use_tc_tiling_on_sc=False),
        scratch_types=[
            pltpu.VMEM((NCH, CH), jnp.int32),
            pltpu.VMEM((MLD, DIM), jnp.float32),
            pltpu.VMEM_SHARED((N, DIM), jnp.float32),
            pltpu.SemaphoreType.DMA,
        ],
    )(msg, idx, zero)


def kernel(x, edge_index, edge_attr, batch, stems, stems_batch, slices_x, lin0_w, lin0_b, net_w1, net_b1, net_w2, net_b2, conv_root, conv_bias, gru_w_ih, gru_w_hh, gru_b_ih, gru_b_hh, lin1_w, lin1_b, lin2_w, lin2_b, s2s_w_ih, s2s_w_hh, s2s_b_ih, s2s_b_hh, lin3_w, lin3_b):
    src = edge_index[0]
    dst = edge_index[1]
    src_r = src.reshape(NW, NCH, CH)
    dst_r = dst.reshape(NW, NCH, CH)
    ea4 = edge_attr.reshape(E4, 16)
    eye4 = jnp.eye(4, dtype=jnp.float32)
    w1x = jnp.kron(eye4, net_w1)
    b1x = jnp.tile(net_b1, 4).reshape(1, 512)
    w2x = jnp.stack([jnp.kron(eye4, net_w2[:, 32 * i:32 * i + 32])
                     for i in range(DIM)]).astype(jnp.bfloat16)
    b2x = jnp.stack([jnp.tile(net_b2[32 * i:32 * i + 32], 4).reshape(1, 128)
                     for i in range(DIM)])
    lane = jnp.arange(128, dtype=jnp.int32)
    eis = jnp.stack([((lane[:, None] // 32 == lane[None, :] // 32)
                      & (lane[:, None] % 32 == i)).astype(jnp.bfloat16)
                     for i in range(DIM)])
    zero_t = jnp.zeros((NPT, DIM), jnp.float32)

    out = pl.pallas_call(
        _lin0_body,
        out_shape=jax.ShapeDtypeStruct((N, DIM), jnp.float32),
    )(x, lin0_w, lin0_b.reshape(1, DIM))

    w_edge = pl.pallas_call(
        _prep_w_body,
        grid=(E4 // RB,),
        in_specs=[
            pl.BlockSpec((RB, 16), lambda i: (i, 0)),
            pl.BlockSpec((16, 512), lambda i: (0, 0)),
            pl.BlockSpec((1, 512), lambda i: (0, 0)),
            pl.BlockSpec((DIM, 512, 128), lambda i: (0, 0, 0)),
            pl.BlockSpec((DIM, 1, 128), lambda i: (0, 0, 0)),
        ],
        out_specs=pl.BlockSpec((DIM, RB, 128), lambda i: (0, i, 0)),
        out_shape=jax.ShapeDtypeStruct((DIM, E4, 128), jnp.bfloat16),
    )(ea4, w1x, b1x, w2x, b2x)

    aggc = _sc_scatter(jnp.ones((E, DIM), jnp.float32), dst_r, zero_t)
    cnt = aggc[0, :, 0:1] + aggc[1, :, 0:1]
    dinv = jnp.broadcast_to(1.0 / jnp.maximum(cnt, 1.0), (N, DIM))

    gru_call = pl.pallas_call(
        _gru_body,
        grid=(10,),
        in_specs=[
            pl.BlockSpec((NCORE, N // 10, DIM), lambda i: (0, i, 0)),
            pl.BlockSpec((N // 10, DIM), lambda i: (i, 0)),
            pl.BlockSpec((N // 10, DIM), lambda i: (i, 0)),
            pl.BlockSpec((DIM, DIM), lambda i: (0, 0)),
            pl.BlockSpec((1, DIM), lambda i: (0, 0)),
            pl.BlockSpec((DIM, 3 * DIM), lambda i: (0, 0)),
            pl.BlockSpec((1, 3 * DIM), lambda i: (0, 0)),
            pl.BlockSpec((DIM, 3 * DIM), lambda i: (0, 0)),
            pl.BlockSpec((1, 3 * DIM), lambda i: (0, 0)),
        ],
        out_specs=pl.BlockSpec((N // 10, DIM), lambda i: (i, 0)),
        out_shape=jax.ShapeDtypeStruct((N, DIM), jnp.float32),
    )

    einsum_call = pl.pallas_call(
        _einsum_body,
        grid=(E4 // RB,),
        in_specs=[
            pl.BlockSpec((RB, 128), lambda i: (i, 0)),
            pl.BlockSpec((DIM, RB, 128), lambda i: (0, i, 0)),
            pl.BlockSpec((DIM, 128, 128), lambda i: (0, 0, 0)),
        ],
        out_specs=pl.BlockSpec((RB, 128), lambda i: (i, 0)),
        out_shape=jax.ShapeDtypeStruct((E4, 128), jnp.float32),
    )

    cr = conv_root
    cb = conv_bias.reshape(1, DIM)
    wih = gru_w_ih.T
    bih = gru_b_ih.reshape(1, 3 * DIM)
    whh = gru_w_hh.T
    bhh = gru_b_hh.reshape(1, 3 * DIM)

    for _ in range(STEPS):
        xj = _sc_gather(out, src_r, (E, DIM), EPW, MLD, CH)
        msg4 = einsum_call(xj.reshape(E4, 128), w_edge, eis)
        agg2 = _sc_scatter(msg4.reshape(E, DIM), dst_r, zero_t)
        out = gru_call(agg2, out, dinv, cr, cb, wih, bih, whh, bhh)

    stem_idx = slices_x[stems_batch] + stems
    sidx = jnp.zeros((S_PAD,), jnp.int32).at[0:500].set(stem_idx)
    srows = _sc_gather(out, sidx.reshape(NW, 1, S_EPW), (S_PAD, DIM), S_EPW, S_EPW, S_EPW)
    per_stem_pad = pl.pallas_call(
        _stem_body,
        out_shape=jax.ShapeDtypeStruct((S_PAD, 105), jnp.float32),
    )(srows, lin1_w, lin1_b.reshape(1, 8 * DIM), lin2_w, lin2_b.reshape(1, 105))
    per_stem_out = per_stem_pad[0:500]

    sout = pl.pallas_call(
        _s2s_body,
        out_shape=jax.ShapeDtypeStruct((B, 1), jnp.float32),
    )(out, batch.reshape(1, N), s2s_w_ih.T, s2s_b_ih.reshape(1, 4 * DIM),
      s2s_w_hh.T, s2s_b_hh.reshape(1, 4 * DIM), lin3_w, lin3_b.reshape(1, 1))

    return (sout, per_stem_out)


# speedup vs baseline: 1.2431x; 1.2431x over previous
import functools

import jax
import jax.numpy as jnp
from jax import lax
from jax.experimental import pallas as pl
from jax.experimental.pallas import tpu as pltpu
from jax.experimental.pallas import tpu_sc as plsc

N = 10000
E = 160000
B = 100
DIM = 32
STEPS = 12
NCORE = 2
NSUB = 16
NW = NCORE * NSUB
EPW = E // NW
CH = 40
NCH = EPW // CH
MLD = 1000
NLD = EPW // MLD
SPL = MLD // CH
NPT = N // NSUB
BE = 640
E4 = E // 4
RB = 1600
S_PAD = 512
S_EPW = S_PAD // NW


def _leaky(v):
    return jax.nn.leaky_relu(v, 0.01)


def _lin0_body(x_ref, w_ref, b_ref, o_ref):
    o_ref[...] = _leaky(jnp.dot(x_ref[...], w_ref[...],
                                preferred_element_type=jnp.float32) + b_ref[...])


def _prep_w_body(ea4_ref, w1x_ref, b1x_ref, w2x_ref, b2x_ref, w_ref):
    he4 = _leaky(jnp.dot(ea4_ref[...], w1x_ref[...],
                         preferred_element_type=jnp.float32) + b1x_ref[...])
    he4b = he4.astype(jnp.bfloat16)
    for i in range(DIM):
        wi = jnp.dot(he4b, w2x_ref[i],
                     preferred_element_type=jnp.float32) + b2x_ref[i]
        w_ref[i] = wi.astype(jnp.bfloat16)


def _einsum_body(xj_ref, w_ref, ei_ref, msg_ref):
    xb = xj_ref[...].astype(jnp.bfloat16)
    acc = jnp.zeros((RB, 128), jnp.float32)
    for i in range(DIM):
        xe = jnp.dot(xb, ei_ref[i], preferred_element_type=jnp.float32)
        acc = acc + xe * w_ref[i].astype(jnp.float32)
    msg_ref[...] = acc


def _gru_body(agg_ref, out_ref, dinv_ref, cr_ref, cb_ref, wih_ref, bih_ref,
              whh_ref, bhh_ref, newo_ref):
    out = out_ref[...]
    agg = (agg_ref[0] + agg_ref[1]) * dinv_ref[...]
    m = _leaky(agg + jnp.dot(out, cr_ref[...],
                             preferred_element_type=jnp.float32) + cb_ref[...])
    gi = jnp.dot(m, wih_ref[...], preferred_element_type=jnp.float32) + bih_ref[...]
    gh = jnp.dot(out, whh_ref[...], preferred_element_type=jnp.float32) + bhh_ref[...]
    r = jax.nn.sigmoid(gi[:, 0:DIM] + gh[:, 0:DIM])
    z = jax.nn.sigmoid(gi[:, DIM:2 * DIM] + gh[:, DIM:2 * DIM])
    n = jnp.tanh(gi[:, 2 * DIM:] + r * gh[:, 2 * DIM:])
    newo = (1.0 - z) * n + z * out
    newo_ref[...] = newo


def _stem_body(rows_ref, w1_ref, b1_ref, w2_ref, b2_ref, o_ref):
    t = _leaky(jnp.dot(rows_ref[...], w1_ref[...],
                       preferred_element_type=jnp.float32) + b1_ref[...])
    o_ref[...] = jnp.dot(t, w2_ref[...],
                         preferred_element_type=jnp.float32) + b2_ref[...]


def _s2s_body(out_ref, bt_ref, wih_ref, bih_ref, whh_ref, bhh_ref,
              w3_ref, b3_ref, sout_ref):
    outT = jnp.transpose(out_ref[...])
    seg = jax.lax.broadcasted_iota(jnp.int32, (B, N), 0) == bt_ref[...]
    segf = seg.astype(jnp.float32)
    q_star = jnp.zeros((B, 2 * DIM), jnp.float32)
    hL = jnp.zeros((B, DIM), jnp.float32)
    cL = jnp.zeros((B, DIM), jnp.float32)
    for _ in range(3):
        gates = (jnp.dot(q_star, wih_ref[...], preferred_element_type=jnp.float32)
                 + bih_ref[...]
                 + jnp.dot(hL, whh_ref[...], preferred_element_type=jnp.float32)
                 + bhh_ref[...])
        ii = gates[:, 0:DIM]
        ff = gates[:, DIM:2 * DIM]
        gg = gates[:, 2 * DIM:3 * DIM]
        oo = gates[:, 3 * DIM:]
        cL = jax.nn.sigmoid(ff) * cL + jax.nn.sigmoid(ii) * jnp.tanh(gg)
        hL = jax.nn.sigmoid(oo) * jnp.tanh(cL)
        qbT = lax.dot_general(jnp.transpose(hL), segf, (((1,), (0,)), ((), ())),
                              preferred_element_type=jnp.float32)
        eT = jnp.sum(outT * qbT, axis=0, keepdims=True)
        em = jnp.where(seg, eT, -jnp.inf)
        mx = jnp.max(em, axis=1, keepdims=True)
        mx = jnp.where(jnp.isfinite(mx), mx, 0.0)
        p = jnp.where(seg, jnp.exp(eT - mx), 0.0)
        s = jnp.sum(p, axis=1, keepdims=True)
        s = jnp.where(s == 0.0, 1.0, s)
        a = p / s
        rvec = lax.dot_general(a, outT, (((1,), (1,)), ((), ())),
                               preferred_element_type=jnp.float32)
        q_star = jnp.concatenate([hL, rvec], axis=1)
    sout_ref[...] = jnp.dot(q_star, w3_ref[...],
                            preferred_element_type=jnp.float32) + b3_ref[...]


def _sc_gather_body(epw, mld, ch, table_hbm, idx_hbm, out_hbm, idx_v, rows_v, sem):
    wid = lax.axis_index("c") * NSUB + lax.axis_index("s")
    nld = epw // mld
    spl = mld // ch
    pltpu.sync_copy(idx_hbm.at[wid], idx_v)

    def round_(g, c):
        def fire(k, c2):
            pltpu.async_copy(table_hbm.at[idx_v.at[g * spl + k]],
                             rows_v.at[pl.ds(k * ch, ch)], sem)
            return c2

        lax.fori_loop(0, spl, fire, 0)

        def drain(k, c2):
            pltpu.make_async_copy(table_hbm.at[idx_v.at[0]],
                                  rows_v.at[pl.ds(0, ch)], sem).wait()
            return c2

        lax.fori_loop(0, spl, drain, 0)
        pltpu.sync_copy(rows_v, out_hbm.at[pl.ds(wid * epw + g * mld, mld)])
        return c

    lax.fori_loop(0, nld, round_, 0)


def _sc_scatter_body(msg_hbm, idx_hbm, zero_hbm, agg_hbm, idx_v, msg_v, agg_sh, sem):
    cid = lax.axis_index("c")
    sid = lax.axis_index("s")
    wid = cid * NSUB + sid
    pltpu.sync_copy(idx_hbm.at[wid], idx_v)
    pltpu.sync_copy(zero_hbm, agg_sh.at[pl.ds(sid * NPT, NPT)])
    plsc.subcore_barrier()

    def body(g, c):
        pltpu.sync_copy(msg_hbm.at[pl.ds(wid * EPW + g * MLD, MLD)], msg_v)

        def sub(k, c2):
            pltpu.sync_copy(msg_v.at[pl.ds(k * CH, CH)],
                            agg_sh.at[idx_v.at[g * SPL + k]], add=True)
            return c2

        lax.fori_loop(0, SPL, sub, 0)
        return c

    lax.fori_loop(0, NLD, body, 0)
    plsc.subcore_barrier()
    pltpu.sync_copy(agg_sh.at[pl.ds(sid * NPT, NPT)],
                    agg_hbm.at[cid, pl.ds(sid * NPT, NPT)])


_MESH = plsc.VectorSubcoreMesh(core_axis_name="c", subcore_axis_name="s")


def _sc_gather(table, idx, rows_out_shape, epw, mld, ch):
    dt = table.dtype
    nch = epw // ch
    return pl.kernel(
        functools.partial(_sc_gather_body, epw, mld, ch),
        out_type=jax.ShapeDtypeStruct(rows_out_shape, dt),
        mesh=_MESH,
        compiler_params=pltpu.CompilerParams(use_tc_tiling_on_sc=False),
        scratch_types=[
            pltpu.VMEM((nch, ch), jnp.int32),
            pltpu.VMEM((mld, DIM), dt),
            pltpu.SemaphoreType.DMA,
        ],
    )(table, idx)


def _sc_scatter(msg, idx, zero):
    return pl.kernel(
        _sc_scatter_body,
        out_type=jax.ShapeDtypeStruct((NCORE, N, DIM), jnp.float32),
        mesh=_MESH,
        compiler_params=pltpu.CompilerParams(use_tc_tiling_on_sc=False),
        scratch_types=[
            pltpu.VMEM((NCH, CH), jnp.int32),
            pltpu.VMEM((MLD, DIM), jnp.float32),
            pltpu.VMEM_SHARED((N, DIM), jnp.float32),
            pltpu.SemaphoreType.DMA,
        ],
    )(msg, idx, zero)


def kernel(x, edge_index, edge_attr, batch, stems, stems_batch, slices_x, lin0_w, lin0_b, net_w1, net_b1, net_w2, net_b2, conv_root, conv_bias, gru_w_ih, gru_w_hh, gru_b_ih, gru_b_hh, lin1_w, lin1_b, lin2_w, lin2_b, s2s_w_ih, s2s_w_hh, s2s_b_ih, s2s_b_hh, lin3_w, lin3_b):
    src = edge_index[0]
    dst = edge_index[1]
    src_r = src.reshape(NW, NCH, CH)
    dst_r = dst.reshape(NW, NCH, CH)
    ea4 = edge_attr.reshape(E4, 16)
    eye4 = jnp.eye(4, dtype=jnp.float32)
    w1x = jnp.kron(eye4, net_w1)
    b1x = jnp.tile(net_b1, 4).reshape(1, 512)
    w2x = jnp.stack([jnp.kron(eye4, net_w2[:, 32 * i:32 * i + 32])
                     for i in range(DIM)]).astype(jnp.bfloat16)
    b2x = jnp.stack([jnp.tile(net_b2[32 * i:32 * i + 32], 4).reshape(1, 128)
                     for i in range(DIM)])
    lane = jnp.arange(128, dtype=jnp.int32)
    eis = jnp.stack([((lane[:, None] // 32 == lane[None, :] // 32)
                      & (lane[:, None] % 32 == i)).astype(jnp.bfloat16)
                     for i in range(DIM)])
    zero_t = jnp.zeros((NPT, DIM), jnp.float32)

    out = pl.pallas_call(
        _lin0_body,
        out_shape=jax.ShapeDtypeStruct((N, DIM), jnp.float32),
    )(x, lin0_w, lin0_b.reshape(1, DIM))

    w_edge = pl.pallas_call(
        _prep_w_body,
        grid=(E4 // RB,),
        in_specs=[
            pl.BlockSpec((RB, 16), lambda i: (i, 0)),
            pl.BlockSpec((16, 512), lambda i: (0, 0)),
            pl.BlockSpec((1, 512), lambda i: (0, 0)),
            pl.BlockSpec((DIM, 512, 128), lambda i: (0, 0, 0)),
            pl.BlockSpec((DIM, 1, 128), lambda i: (0, 0, 0)),
        ],
        out_specs=pl.BlockSpec((DIM, RB, 128), lambda i: (0, i, 0)),
        out_shape=jax.ShapeDtypeStruct((DIM, E4, 128), jnp.bfloat16),
    )(ea4, w1x, b1x, w2x, b2x)

    aggc = _sc_scatter(jnp.ones((E, DIM), jnp.float32), dst_r, zero_t)
    cnt = aggc[0, :, 0:1] + aggc[1, :, 0:1]
    dinv = jnp.broadcast_to(1.0 / jnp.maximum(cnt, 1.0), (N, DIM))

    gru_call = pl.pallas_call(
        _gru_body,
        grid=(10,),
        in_specs=[
            pl.BlockSpec((NCORE, N // 10, DIM), lambda i: (0, i, 0)),
            pl.BlockSpec((N // 10, DIM), lambda i: (i, 0)),
            pl.BlockSpec((N // 10, DIM), lambda i: (i, 0)),
            pl.BlockSpec((DIM, DIM), lambda i: (0, 0)),
            pl.BlockSpec((1, DIM), lambda i: (0, 0)),
            pl.BlockSpec((DIM, 3 * DIM), lambda i: (0, 0)),
            pl.BlockSpec((1, 3 * DIM), lambda i: (0, 0)),
            pl.BlockSpec((DIM, 3 * DIM), lambda i: (0, 0)),
            pl.BlockSpec((1, 3 * DIM), lambda i: (0, 0)),
        ],
        out_specs=pl.BlockSpec((N // 10, DIM), lambda i: (i, 0)),
        out_shape=jax.ShapeDtypeStruct((N, DIM), jnp.float32),
    )

    einsum_call = pl.pallas_call(
        _einsum_body,
        grid=(E4 // RB,),
        in_specs=[
            pl.BlockSpec((RB, 128), lambda i: (i, 0)),
            pl.BlockSpec((DIM, RB, 128), lambda i: (0, i, 0)),
            pl.BlockSpec((DIM, 128, 128), lambda i: (0, 0, 0)),
        ],
        out_specs=pl.BlockSpec((RB, 128), lambda i: (i, 0)),
        out_shape=jax.ShapeDtypeStruct((E4, 128), jnp.float32),
    )

    cr = conv_root
    cb = conv_bias.reshape(1, DIM)
    wih = gru_w_ih.T
    bih = gru_b_ih.reshape(1, 3 * DIM)
    whh = gru_w_hh.T
    bhh = gru_b_hh.reshape(1, 3 * DIM)

    for _ in range(STEPS):
        xj = _sc_gather(out, src_r, (E, DIM), EPW, MLD, CH)
        msg4 = einsum_call(xj.reshape(E4, 128), w_edge, eis)
        agg2 = _sc_scatter(msg4.reshape(E, DIM), dst_r, zero_t)
        out = gru_call(agg2, out, dinv, cr, cb, wih, bih, whh, bhh)

    stem_idx = slices_x[stems_batch] + stems
    sidx = jnp.zeros((S_PAD,), jnp.int32).at[0:500].set(stem_idx)
    srows = _sc_gather(out, sidx.reshape(NW, 1, S_EPW), (S_PAD, DIM), S_EPW, S_EPW, S_EPW)
    per_stem_pad = pl.pallas_call(
        _stem_body,
        out_shape=jax.ShapeDtypeStruct((S_PAD, 105), jnp.float32),
    )(srows, lin1_w, lin1_b.reshape(1, 8 * DIM), lin2_w, lin2_b.reshape(1, 105))
    per_stem_out = per_stem_pad[0:500]

    sout = pl.pallas_call(
        _s2s_body,
        out_shape=jax.ShapeDtypeStruct((B, 1), jnp.float32),
    )(out, batch.reshape(1, N), s2s_w_ih.T, s2s_b_ih.reshape(1, 4 * DIM),
      s2s_w_hh.T, s2s_b_hh.reshape(1, 4 * DIM), lin3_w, lin3_b.reshape(1, 1))

    return (sout, per_stem_out)


# speedup vs baseline: 1.2438x; 1.0005x over previous
import functools

import jax
import jax.numpy as jnp
from jax import lax
from jax.experimental import pallas as pl
from jax.experimental.pallas import tpu as pltpu
from jax.experimental.pallas import tpu_sc as plsc

N = 10000
E = 160000
B = 100
DIM = 32
STEPS = 12
NCORE = 2
NSUB = 16
NW = NCORE * NSUB
EPW = E // NW
CH = 40
NCH = EPW // CH
MLD = 1000
NLD = EPW // MLD
SPL = MLD // CH
NPT = N // NSUB
BE = 640
E4 = E // 4
RB = 1600
S_PAD = 512
S_EPW = S_PAD // NW


def _leaky(v):
    return jax.nn.leaky_relu(v, 0.01)


def _lin0_body(x_ref, w_ref, b_ref, o_ref):
    o_ref[...] = _leaky(jnp.dot(x_ref[...], w_ref[...],
                                preferred_element_type=jnp.float32) + b_ref[...])


def _prep_w_body(ea4_ref, w1x_ref, b1x_ref, w2x_ref, b2x_ref, w_ref):
    he4 = _leaky(jnp.dot(ea4_ref[...], w1x_ref[...],
                         preferred_element_type=jnp.float32) + b1x_ref[...])
    he4b = he4.astype(jnp.bfloat16)
    for i in range(DIM):
        wi = jnp.dot(he4b, w2x_ref[i],
                     preferred_element_type=jnp.float32) + b2x_ref[i]
        w_ref[i] = wi.astype(jnp.bfloat16)


def _einsum_body(xj_ref, w_ref, ei_ref, msg_ref):
    xb = xj_ref[...].astype(jnp.bfloat16)
    acc = jnp.zeros((RB, 128), jnp.float32)
    for i in range(DIM):
        xe = jnp.dot(xb, ei_ref[i], preferred_element_type=jnp.float32)
        acc = acc + (xe.astype(jnp.bfloat16) * w_ref[i]).astype(jnp.float32)
    msg_ref[...] = acc


def _gru_body(agg_ref, out_ref, dinv_ref, cr_ref, cb_ref, wih_ref, bih_ref,
              whh_ref, bhh_ref, newo_ref):
    out = out_ref[...]
    agg = (agg_ref[0] + agg_ref[1]) * dinv_ref[...]
    m = _leaky(agg + jnp.dot(out, cr_ref[...],
                             preferred_element_type=jnp.float32) + cb_ref[...])
    gi = jnp.dot(m, wih_ref[...], preferred_element_type=jnp.float32) + bih_ref[...]
    gh = jnp.dot(out, whh_ref[...], preferred_element_type=jnp.float32) + bhh_ref[...]
    r = jax.nn.sigmoid(gi[:, 0:DIM] + gh[:, 0:DIM])
    z = jax.nn.sigmoid(gi[:, DIM:2 * DIM] + gh[:, DIM:2 * DIM])
    n = jnp.tanh(gi[:, 2 * DIM:] + r * gh[:, 2 * DIM:])
    newo = (1.0 - z) * n + z * out
    newo_ref[...] = newo


def _stem_body(rows_ref, w1_ref, b1_ref, w2_ref, b2_ref, o_ref):
    t = _leaky(jnp.dot(rows_ref[...], w1_ref[...],
                       preferred_element_type=jnp.float32) + b1_ref[...])
    o_ref[...] = jnp.dot(t, w2_ref[...],
                         preferred_element_type=jnp.float32) + b2_ref[...]


def _s2s_body(out_ref, bt_ref, wih_ref, bih_ref, whh_ref, bhh_ref,
              w3_ref, b3_ref, sout_ref):
    outT = jnp.transpose(out_ref[...])
    seg = jax.lax.broadcasted_iota(jnp.int32, (B, N), 0) == bt_ref[...]
    segf = seg.astype(jnp.float32)
    q_star = jnp.zeros((B, 2 * DIM), jnp.float32)
    hL = jnp.zeros((B, DIM), jnp.float32)
    cL = jnp.zeros((B, DIM), jnp.float32)
    for _ in range(3):
        gates = (jnp.dot(q_star, wih_ref[...], preferred_element_type=jnp.float32)
                 + bih_ref[...]
                 + jnp.dot(hL, whh_ref[...], preferred_element_type=jnp.float32)
                 + bhh_ref[...])
        ii = gates[:, 0:DIM]
        ff = gates[:, DIM:2 * DIM]
        gg = gates[:, 2 * DIM:3 * DIM]
        oo = gates[:, 3 * DIM:]
        cL = jax.nn.sigmoid(ff) * cL + jax.nn.sigmoid(ii) * jnp.tanh(gg)
        hL = jax.nn.sigmoid(oo) * jnp.tanh(cL)
        qbT = lax.dot_general(jnp.transpose(hL), segf, (((1,), (0,)), ((), ())),
                              preferred_element_type=jnp.float32)
        eT = jnp.sum(outT * qbT, axis=0, keepdims=True)
        em = jnp.where(seg, eT, -jnp.inf)
        mx = jnp.max(em, axis=1, keepdims=True)
        mx = jnp.where(jnp.isfinite(mx), mx, 0.0)
        p = jnp.where(seg, jnp.exp(eT - mx), 0.0)
        s = jnp.sum(p, axis=1, keepdims=True)
        s = jnp.where(s == 0.0, 1.0, s)
        a = p / s
        rvec = lax.dot_general(a, outT, (((1,), (1,)), ((), ())),
                               preferred_element_type=jnp.float32)
        q_star = jnp.concatenate([hL, rvec], axis=1)
    sout_ref[...] = jnp.dot(q_star, w3_ref[...],
                            preferred_element_type=jnp.float32) + b3_ref[...]


def _sc_gather_body(epw, mld, ch, table_hbm, idx_hbm, out_hbm, idx_v, rows_v, sem):
    wid = lax.axis_index("c") * NSUB + lax.axis_index("s")
    nld = epw // mld
    spl = mld // ch
    pltpu.sync_copy(idx_hbm.at[wid], idx_v)

    def round_(g, c):
        def fire(k, c2):
            pltpu.async_copy(table_hbm.at[idx_v.at[g * spl + k]],
                             rows_v.at[pl.ds(k * ch, ch)], sem)
            return c2

        lax.fori_loop(0, spl, fire, 0)

        def drain(k, c2):
            pltpu.make_async_copy(table_hbm.at[idx_v.at[0]],
                                  rows_v.at[pl.ds(0, ch)], sem).wait()
            return c2

        lax.fori_loop(0, spl, drain, 0)
        pltpu.sync_copy(rows_v, out_hbm.at[pl.ds(wid * epw + g * mld, mld)])
        return c

    lax.fori_loop(0, nld, round_, 0)


def _sc_scatter_body(msg_hbm, idx_hbm, zero_hbm, agg_hbm, idx_v, msg_v, agg_sh, sem):
    cid = lax.axis_index("c")
    sid = lax.axis_index("s")
    wid = cid * NSUB + sid
    pltpu.sync_copy(idx_hbm.at[wid], idx_v)
    pltpu.sync_copy(zero_hbm, agg_sh.at[pl.ds(sid * NPT, NPT)])
    plsc.subcore_barrier()

    def body(g, c):
        pltpu.sync_copy(msg_hbm.at[pl.ds(wid * EPW + g * MLD, MLD)], msg_v)

        def sub(k, c2):
            pltpu.sync_copy(msg_v.at[pl.ds(k * CH, CH)],
                            agg_sh.at[idx_v.at[g * SPL + k]], add=True)
            return c2

        lax.fori_loop(0, SPL, sub, 0)
        return c

    lax.fori_loop(0, NLD, body, 0)
    plsc.subcore_barrier()
    pltpu.sync_copy(agg_sh.at[pl.ds(sid * NPT, NPT)],
                    agg_hbm.at[cid, pl.ds(sid * NPT, NPT)])


_MESH = plsc.VectorSubcoreMesh(core_axis_name="c", subcore_axis_name="s")


def _sc_gather(table, idx, rows_out_shape, epw, mld, ch):
    dt = table.dtype
    nch = epw // ch
    return pl.kernel(
        functools.partial(_sc_gather_body, epw, mld, ch),
        out_type=jax.ShapeDtypeStruct(rows_out_shape, dt),
        mesh=_MESH,
        compiler_params=pltpu.CompilerParams(use_tc_tiling_on_sc=False),
        scratch_types=[
            pltpu.VMEM((nch, ch), jnp.int32),
            pltpu.VMEM((mld, DIM), dt),
            pltpu.SemaphoreType.DMA,
        ],
    )(table, idx)


def _sc_scatter(msg, idx, zero):
    return pl.kernel(
        _sc_scatter_body,
        out_type=jax.ShapeDtypeStruct((NCORE, N, DIM), jnp.float32),
        mesh=_MESH,
        compiler_params=pltpu.CompilerParams(use_tc_tiling_on_sc=False),
        scratch_types=[
            pltpu.VMEM((NCH, CH), jnp.int32),
            pltpu.VMEM((MLD, DIM), jnp.float32),
            pltpu.VMEM_SHARED((N, DIM), jnp.float32),
            pltpu.SemaphoreType.DMA,
        ],
    )(msg, idx, zero)


def kernel(x, edge_index, edge_attr, batch, stems, stems_batch, slices_x, lin0_w, lin0_b, net_w1, net_b1, net_w2, net_b2, conv_root, conv_bias, gru_w_ih, gru_w_hh, gru_b_ih, gru_b_hh, lin1_w, lin1_b, lin2_w, lin2_b, s2s_w_ih, s2s_w_hh, s2s_b_ih, s2s_b_hh, lin3_w, lin3_b):
    src = edge_index[0]
    dst = edge_index[1]
    src_r = src.reshape(NW, NCH, CH)
    dst_r = dst.reshape(NW, NCH, CH)
    ea4 = edge_attr.reshape(E4, 16)
    eye4 = jnp.eye(4, dtype=jnp.float32)
    w1x = jnp.kron(eye4, net_w1)
    b1x = jnp.tile(net_b1, 4).reshape(1, 512)
    w2x = jnp.stack([jnp.kron(eye4, net_w2[:, 32 * i:32 * i + 32])
                     for i in range(DIM)]).astype(jnp.bfloat16)
    b2x = jnp.stack([jnp.tile(net_b2[32 * i:32 * i + 32], 4).reshape(1, 128)
                     for i in range(DIM)])
    lane = jnp.arange(128, dtype=jnp.int32)
    eis = jnp.stack([((lane[:, None] // 32 == lane[None, :] // 32)
                      & (lane[:, None] % 32 == i)).astype(jnp.bfloat16)
                     for i in range(DIM)])
    zero_t = jnp.zeros((NPT, DIM), jnp.float32)

    out = pl.pallas_call(
        _lin0_body,
        out_shape=jax.ShapeDtypeStruct((N, DIM), jnp.float32),
    )(x, lin0_w, lin0_b.reshape(1, DIM))

    w_edge = pl.pallas_call(
        _prep_w_body,
        grid=(E4 // RB,),
        in_specs=[
            pl.BlockSpec((RB, 16), lambda i: (i, 0)),
            pl.BlockSpec((16, 512), lambda i: (0, 0)),
            pl.BlockSpec((1, 512), lambda i: (0, 0)),
            pl.BlockSpec((DIM, 512, 128), lambda i: (0, 0, 0)),
            pl.BlockSpec((DIM, 1, 128), lambda i: (0, 0, 0)),
        ],
        out_specs=pl.BlockSpec((DIM, RB, 128), lambda i: (0, i, 0)),
        out_shape=jax.ShapeDtypeStruct((DIM, E4, 128), jnp.bfloat16),
    )(ea4, w1x, b1x, w2x, b2x)

    aggc = _sc_scatter(jnp.ones((E, DIM), jnp.float32), dst_r, zero_t)
    cnt = aggc[0, :, 0:1] + aggc[1, :, 0:1]
    dinv = jnp.broadcast_to(1.0 / jnp.maximum(cnt, 1.0), (N, DIM))

    gru_call = pl.pallas_call(
        _gru_body,
        grid=(10,),
        in_specs=[
            pl.BlockSpec((NCORE, N // 10, DIM), lambda i: (0, i, 0)),
            pl.BlockSpec((N // 10, DIM), lambda i: (i, 0)),
            pl.BlockSpec((N // 10, DIM), lambda i: (i, 0)),
            pl.BlockSpec((DIM, DIM), lambda i: (0, 0)),
            pl.BlockSpec((1, DIM), lambda i: (0, 0)),
            pl.BlockSpec((DIM, 3 * DIM), lambda i: (0, 0)),
            pl.BlockSpec((1, 3 * DIM), lambda i: (0, 0)),
            pl.BlockSpec((DIM, 3 * DIM), lambda i: (0, 0)),
            pl.BlockSpec((1, 3 * DIM), lambda i: (0, 0)),
        ],
        out_specs=pl.BlockSpec((N // 10, DIM), lambda i: (i, 0)),
        out_shape=jax.ShapeDtypeStruct((N, DIM), jnp.float32),
    )

    einsum_call = pl.pallas_call(
        _einsum_body,
        grid=(E4 // RB,),
        in_specs=[
            pl.BlockSpec((RB, 128), lambda i: (i, 0)),
            pl.BlockSpec((DIM, RB, 128), lambda i: (0, i, 0)),
            pl.BlockSpec((DIM, 128, 128), lambda i: (0, 0, 0)),
        ],
        out_specs=pl.BlockSpec((RB, 128), lambda i: (i, 0)),
        out_shape=jax.ShapeDtypeStruct((E4, 128), jnp.float32),
    )

    cr = conv_root
    cb = conv_bias.reshape(1, DIM)
    wih = gru_w_ih.T
    bih = gru_b_ih.reshape(1, 3 * DIM)
    whh = gru_w_hh.T
    bhh = gru_b_hh.reshape(1, 3 * DIM)

    for _ in range(STEPS):
        xj = _sc_gather(out, src_r, (E, DIM), EPW, MLD, CH)
        msg4 = einsum_call(xj.reshape(E4, 128), w_edge, eis)
        agg2 = _sc_scatter(msg4.reshape(E, DIM), dst_r, zero_t)
        out = gru_call(agg2, out, dinv, cr, cb, wih, bih, whh, bhh)

    stem_idx = slices_x[stems_batch] + stems
    sidx = jnp.zeros((S_PAD,), jnp.int32).at[0:500].set(stem_idx)
    srows = _sc_gather(out, sidx.reshape(NW, 1, S_EPW), (S_PAD, DIM), S_EPW, S_EPW, S_EPW)
    per_stem_pad = pl.pallas_call(
        _stem_body,
        out_shape=jax.ShapeDtypeStruct((S_PAD, 105), jnp.float32),
    )(srows, lin1_w, lin1_b.reshape(1, 8 * DIM), lin2_w, lin2_b.reshape(1, 105))
    per_stem_out = per_stem_pad[0:500]

    sout = pl.pallas_call(
        _s2s_body,
        out_shape=jax.ShapeDtypeStruct((B, 1), jnp.float32),
    )(out, batch.reshape(1, N), s2s_w_ih.T, s2s_b_ih.reshape(1, 4 * DIM),
      s2s_w_hh.T, s2s_b_hh.reshape(1, 4 * DIM), lin3_w, lin3_b.reshape(1, 1))

    return (sout, per_stem_out)


# speedup vs baseline: 1.2796x; 1.0288x over previous
import functools

import jax
import jax.numpy as jnp
from jax import lax
from jax.experimental import pallas as pl
from jax.experimental.pallas import tpu as pltpu
from jax.experimental.pallas import tpu_sc as plsc

N = 10000
E = 160000
B = 100
DIM = 32
STEPS = 12
NCORE = 2
NSUB = 16
NW = NCORE * NSUB
EPW = E // NW
CH = 40
NCH = EPW // CH
MLD = 1000
NLD = EPW // MLD
SPL = MLD // CH
NPT = N // NSUB
BE = 640
E4 = E // 4
RB = 1600
S_PAD = 512
S_EPW = S_PAD // NW


def _leaky(v):
    return jax.nn.leaky_relu(v, 0.01)


def _lin0_body(x_ref, w_ref, b_ref, o_ref):
    o_ref[...] = _leaky(jnp.dot(x_ref[...], w_ref[...],
                                preferred_element_type=jnp.float32) + b_ref[...])


def _prep_w_body(ea4_ref, w1x_ref, b1x_ref, w2x_ref, b2x_ref, w_ref):
    he4 = _leaky(jnp.dot(ea4_ref[...], w1x_ref[...],
                         preferred_element_type=jnp.float32) + b1x_ref[...])
    he4b = he4.astype(jnp.bfloat16)
    for i in range(DIM):
        wi = jnp.dot(he4b, w2x_ref[i],
                     preferred_element_type=jnp.float32) + b2x_ref[i]
        w_ref[i] = wi.astype(jnp.bfloat16)


def _einsum_body(xj_ref, w_ref, ei_ref, msg_ref):
    xb = xj_ref[...].astype(jnp.bfloat16)
    acc = jnp.zeros((RB, 128), jnp.float32)
    for i in range(DIM):
        xe = jnp.dot(xb, ei_ref[i], preferred_element_type=jnp.float32)
        acc = acc + xe * w_ref[i].astype(jnp.float32)
    msg_ref[...] = acc


def _gru_body(agg_ref, out_ref, dinv_ref, cr_ref, cb_ref, wih_ref, bih_ref,
              whh_ref, bhh_ref, newo_ref):
    out = out_ref[...]
    agg = (agg_ref[0] + agg_ref[1]) * dinv_ref[...]
    m = _leaky(agg + jnp.dot(out, cr_ref[...],
                             preferred_element_type=jnp.float32) + cb_ref[...])
    gi = jnp.dot(m, wih_ref[...], preferred_element_type=jnp.float32) + bih_ref[...]
    gh = jnp.dot(out, whh_ref[...], preferred_element_type=jnp.float32) + bhh_ref[...]
    r = jax.nn.sigmoid(gi[:, 0:DIM] + gh[:, 0:DIM])
    z = jax.nn.sigmoid(gi[:, DIM:2 * DIM] + gh[:, DIM:2 * DIM])
    n = jnp.tanh(gi[:, 2 * DIM:] + r * gh[:, 2 * DIM:])
    newo = (1.0 - z) * n + z * out
    newo_ref[...] = newo


def _stem_body(rows_ref, w1_ref, b1_ref, w2_ref, b2_ref, o_ref):
    t = _leaky(jnp.dot(rows_ref[...], w1_ref[...],
                       preferred_element_type=jnp.float32) + b1_ref[...])
    o_ref[...] = jnp.dot(t, w2_ref[...],
                         preferred_element_type=jnp.float32) + b2_ref[...]


def _s2s_body(out_ref, bt_ref, wih_ref, bih_ref, whh_ref, bhh_ref,
              w3_ref, b3_ref, sout_ref):
    outT = jnp.transpose(out_ref[...])
    seg = jax.lax.broadcasted_iota(jnp.int32, (B, N), 0) == bt_ref[...]
    segf = seg.astype(jnp.float32)
    q_star = jnp.zeros((B, 2 * DIM), jnp.float32)
    hL = jnp.zeros((B, DIM), jnp.float32)
    cL = jnp.zeros((B, DIM), jnp.float32)
    for _ in range(3):
        gates = (jnp.dot(q_star, wih_ref[...], preferred_element_type=jnp.float32)
                 + bih_ref[...]
                 + jnp.dot(hL, whh_ref[...], preferred_element_type=jnp.float32)
                 + bhh_ref[...])
        ii = gates[:, 0:DIM]
        ff = gates[:, DIM:2 * DIM]
        gg = gates[:, 2 * DIM:3 * DIM]
        oo = gates[:, 3 * DIM:]
        cL = jax.nn.sigmoid(ff) * cL + jax.nn.sigmoid(ii) * jnp.tanh(gg)
        hL = jax.nn.sigmoid(oo) * jnp.tanh(cL)
        qbT = lax.dot_general(jnp.transpose(hL), segf, (((1,), (0,)), ((), ())),
                              preferred_element_type=jnp.float32)
        eT = jnp.sum(outT * qbT, axis=0, keepdims=True)
        em = jnp.where(seg, eT, -jnp.inf)
        mx = jnp.max(em, axis=1, keepdims=True)
        mx = jnp.where(jnp.isfinite(mx), mx, 0.0)
        p = jnp.where(seg, jnp.exp(eT - mx), 0.0)
        s = jnp.sum(p, axis=1, keepdims=True)
        s = jnp.where(s == 0.0, 1.0, s)
        a = p / s
        rvec = lax.dot_general(a, outT, (((1,), (1,)), ((), ())),
                               preferred_element_type=jnp.float32)
        q_star = jnp.concatenate([hL, rvec], axis=1)
    sout_ref[...] = jnp.dot(q_star, w3_ref[...],
                            preferred_element_type=jnp.float32) + b3_ref[...]


def _sc_gather_body(epw, mld, ch, table_hbm, idx_hbm, out_hbm, idx_v, rows_v, sem):
    wid = lax.axis_index("c") * NSUB + lax.axis_index("s")
    nld = epw // mld
    spl = mld // ch
    pltpu.sync_copy(idx_hbm.at[wid], idx_v)

    def round_(g, c):
        def fire(k, c2):
            pltpu.async_copy(table_hbm.at[idx_v.at[g * spl + k]],
                             rows_v.at[pl.ds(k * ch, ch)], sem)
            return c2

        lax.fori_loop(0, spl, fire, 0)

        def drain(k, c2):
            pltpu.make_async_copy(table_hbm.at[idx_v.at[0]],
                                  rows_v.at[pl.ds(0, ch)], sem).wait()
            return c2

        lax.fori_loop(0, spl, drain, 0)
        pltpu.sync_copy(rows_v, out_hbm.at[pl.ds(wid * epw + g * mld, mld)])
        return c

    lax.fori_loop(0, nld, round_, 0)


def _sc_scatter_body(msg_hbm, idx_hbm, zero_hbm, agg_hbm, idx_v, msg_v, agg_sh, sem):
    cid = lax.axis_index("c")
    sid = lax.axis_index("s")
    wid = cid * NSUB + sid
    pltpu.sync_copy(idx_hbm.at[wid], idx_v)
    pltpu.sync_copy(zero_hbm, agg_sh.at[pl.ds(sid * NPT, NPT)])
    plsc.subcore_barrier()

    def body(g, c):
        pltpu.sync_copy(msg_hbm.at[pl.ds(wid * EPW + g * MLD, MLD)], msg_v)

        def sub(k, c2):
            pltpu.async_copy(msg_v.at[pl.ds(k * CH, CH)],
                             agg_sh.at[idx_v.at[g * SPL + k]], sem, add=True)
            return c2

        lax.fori_loop(0, SPL, sub, 0)

        def sdrain(k, c2):
            pltpu.make_async_copy(msg_v.at[pl.ds(0, CH)],
                                  agg_sh.at[idx_v.at[0]], sem).wait()
            return c2

        lax.fori_loop(0, SPL, sdrain, 0)
        return c

    lax.fori_loop(0, NLD, body, 0)
    plsc.subcore_barrier()
    pltpu.sync_copy(agg_sh.at[pl.ds(sid * NPT, NPT)],
                    agg_hbm.at[cid, pl.ds(sid * NPT, NPT)])


_MESH = plsc.VectorSubcoreMesh(core_axis_name="c", subcore_axis_name="s")


def _sc_gather(table, idx, rows_out_shape, epw, mld, ch):
    dt = table.dtype
    nch = epw // ch
    return pl.kernel(
        functools.partial(_sc_gather_body, epw, mld, ch),
        out_type=jax.ShapeDtypeStruct(rows_out_shape, dt),
        mesh=_MESH,
        compiler_params=pltpu.CompilerParams(use_tc_tiling_on_sc=False),
        scratch_types=[
            pltpu.VMEM((nch, ch), jnp.int32),
            pltpu.VMEM((mld, DIM), dt),
            pltpu.SemaphoreType.DMA,
        ],
    )(table, idx)


def _sc_scatter(msg, idx, zero):
    return pl.kernel(
        _sc_scatter_body,
        out_type=jax.ShapeDtypeStruct((NCORE, N, DIM), jnp.float32),
        mesh=_MESH,
        compiler_params=pltpu.CompilerParams(use_tc_tiling_on_sc=False),
        scratch_types=[
            pltpu.VMEM((NCH, CH), jnp.int32),
            pltpu.VMEM((MLD, DIM), jnp.float32),
            pltpu.VMEM_SHARED((N, DIM), jnp.float32),
            pltpu.SemaphoreType.DMA,
        ],
    )(msg, idx, zero)


def kernel(x, edge_index, edge_attr, batch, stems, stems_batch, slices_x, lin0_w, lin0_b, net_w1, net_b1, net_w2, net_b2, conv_root, conv_bias, gru_w_ih, gru_w_hh, gru_b_ih, gru_b_hh, lin1_w, lin1_b, lin2_w, lin2_b, s2s_w_ih, s2s_w_hh, s2s_b_ih, s2s_b_hh, lin3_w, lin3_b):
    src = edge_index[0]
    dst = edge_index[1]
    src_r = src.reshape(NW, NCH, CH)
    dst_r = dst.reshape(NW, NCH, CH)
    ea4 = edge_attr.reshape(E4, 16)
    eye4 = jnp.eye(4, dtype=jnp.float32)
    w1x = jnp.kron(eye4, net_w1)
    b1x = jnp.tile(net_b1, 4).reshape(1, 512)
    w2x = jnp.stack([jnp.kron(eye4, net_w2[:, 32 * i:32 * i + 32])
                     for i in range(DIM)]).astype(jnp.bfloat16)
    b2x = jnp.stack([jnp.tile(net_b2[32 * i:32 * i + 32], 4).reshape(1, 128)
                     for i in range(DIM)])
    lane = jnp.arange(128, dtype=jnp.int32)
    eis = jnp.stack([((lane[:, None] // 32 == lane[None, :] // 32)
                      & (lane[:, None] % 32 == i)).astype(jnp.bfloat16)
                     for i in range(DIM)])
    zero_t = jnp.zeros((NPT, DIM), jnp.float32)

    out = pl.pallas_call(
        _lin0_body,
        out_shape=jax.ShapeDtypeStruct((N, DIM), jnp.float32),
    )(x, lin0_w, lin0_b.reshape(1, DIM))

    w_edge = pl.pallas_call(
        _prep_w_body,
        grid=(E4 // RB,),
        in_specs=[
            pl.BlockSpec((RB, 16), lambda i: (i, 0)),
            pl.BlockSpec((16, 512), lambda i: (0, 0)),
            pl.BlockSpec((1, 512), lambda i: (0, 0)),
            pl.BlockSpec((DIM, 512, 128), lambda i: (0, 0, 0)),
            pl.BlockSpec((DIM, 1, 128), lambda i: (0, 0, 0)),
        ],
        out_specs=pl.BlockSpec((DIM, RB, 128), lambda i: (0, i, 0)),
        out_shape=jax.ShapeDtypeStruct((DIM, E4, 128), jnp.bfloat16),
    )(ea4, w1x, b1x, w2x, b2x)

    aggc = _sc_scatter(jnp.ones((E, DIM), jnp.float32), dst_r, zero_t)
    cnt = aggc[0, :, 0:1] + aggc[1, :, 0:1]
    dinv = jnp.broadcast_to(1.0 / jnp.maximum(cnt, 1.0), (N, DIM))

    gru_call = pl.pallas_call(
        _gru_body,
        grid=(10,),
        in_specs=[
            pl.BlockSpec((NCORE, N // 10, DIM), lambda i: (0, i, 0)),
            pl.BlockSpec((N // 10, DIM), lambda i: (i, 0)),
            pl.BlockSpec((N // 10, DIM), lambda i: (i, 0)),
            pl.BlockSpec((DIM, DIM), lambda i: (0, 0)),
            pl.BlockSpec((1, DIM), lambda i: (0, 0)),
            pl.BlockSpec((DIM, 3 * DIM), lambda i: (0, 0)),
            pl.BlockSpec((1, 3 * DIM), lambda i: (0, 0)),
            pl.BlockSpec((DIM, 3 * DIM), lambda i: (0, 0)),
            pl.BlockSpec((1, 3 * DIM), lambda i: (0, 0)),
        ],
        out_specs=pl.BlockSpec((N // 10, DIM), lambda i: (i, 0)),
        out_shape=jax.ShapeDtypeStruct((N, DIM), jnp.float32),
    )

    einsum_call = pl.pallas_call(
        _einsum_body,
        grid=(E4 // RB,),
        in_specs=[
            pl.BlockSpec((RB, 128), lambda i: (i, 0)),
            pl.BlockSpec((DIM, RB, 128), lambda i: (0, i, 0)),
            pl.BlockSpec((DIM, 128, 128), lambda i: (0, 0, 0)),
        ],
        out_specs=pl.BlockSpec((RB, 128), lambda i: (i, 0)),
        out_shape=jax.ShapeDtypeStruct((E4, 128), jnp.float32),
    )

    cr = conv_root
    cb = conv_bias.reshape(1, DIM)
    wih = gru_w_ih.T
    bih = gru_b_ih.reshape(1, 3 * DIM)
    whh = gru_w_hh.T
    bhh = gru_b_hh.reshape(1, 3 * DIM)

    for _ in range(STEPS):
        xj = _sc_gather(out, src_r, (E, DIM), EPW, MLD, CH)
        msg4 = einsum_call(xj.reshape(E4, 128), w_edge, eis)
        agg2 = _sc_scatter(msg4.reshape(E, DIM), dst_r, zero_t)
        out = gru_call(agg2, out, dinv, cr, cb, wih, bih, whh, bhh)

    stem_idx = slices_x[stems_batch] + stems
    sidx = jnp.zeros((S_PAD,), jnp.int32).at[0:500].set(stem_idx)
    srows = _sc_gather(out, sidx.reshape(NW, 1, S_EPW), (S_PAD, DIM), S_EPW, S_EPW, S_EPW)
    per_stem_pad = pl.pallas_call(
        _stem_body,
        out_shape=jax.ShapeDtypeStruct((S_PAD, 105), jnp.float32),
    )(srows, lin1_w, lin1_b.reshape(1, 8 * DIM), lin2_w, lin2_b.reshape(1, 105))
    per_stem_out = per_stem_pad[0:500]

    sout = pl.pallas_call(
        _s2s_body,
        out_shape=jax.ShapeDtypeStruct((B, 1), jnp.float32),
    )(out, batch.reshape(1, N), s2s_w_ih.T, s2s_b_ih.reshape(1, 4 * DIM),
      s2s_w_hh.T, s2s_b_hh.reshape(1, 4 * DIM), lin3_w, lin3_b.reshape(1, 1))

    return (sout, per_stem_out)


# speedup vs baseline: 1.2890x; 1.0074x over previous
import functools

import jax
import jax.numpy as jnp
from jax import lax
from jax.experimental import pallas as pl
from jax.experimental.pallas import tpu as pltpu
from jax.experimental.pallas import tpu_sc as plsc

N = 10000
E = 160000
B = 100
DIM = 32
STEPS = 12
NCORE = 2
NSUB = 16
NW = NCORE * NSUB
EPW = E // NW
CH = 40
NCH = EPW // CH
MLD = 1000
NLD = EPW // MLD
SPL = MLD // CH
NPT = N // NSUB
BE = 640
E4 = E // 4
RB = 1600
S_PAD = 512
S_EPW = S_PAD // NW


def _leaky(v):
    return jax.nn.leaky_relu(v, 0.01)


def _lin0_body(x_ref, w_ref, b_ref, o_ref):
    o_ref[...] = _leaky(jnp.dot(x_ref[...], w_ref[...],
                                preferred_element_type=jnp.float32) + b_ref[...])


def _prep_w_body(ea4_ref, w1x_ref, b1x_ref, w2x_ref, b2x_ref, w_ref):
    he4 = _leaky(jnp.dot(ea4_ref[...], w1x_ref[...],
                         preferred_element_type=jnp.float32) + b1x_ref[...])
    he4b = he4.astype(jnp.bfloat16)
    for i in range(DIM):
        wi = jnp.dot(he4b, w2x_ref[i],
                     preferred_element_type=jnp.float32) + b2x_ref[i]
        w_ref[i] = wi.astype(jnp.bfloat16)


def _einsum_body(xj_ref, w_ref, ei_ref, msg_ref):
    xb = xj_ref[...].astype(jnp.bfloat16)
    acc = jnp.zeros((RB, 128), jnp.float32)
    for i in range(DIM):
        xe = jnp.dot(xb, ei_ref[i], preferred_element_type=jnp.float32)
        acc = acc + xe * w_ref[i].astype(jnp.float32)
    msg_ref[...] = acc


def _gru_body(agg_ref, out_ref, dinv_ref, cr_ref, cb_ref, wih_ref, bih_ref,
              whh_ref, bhh_ref, newo_ref):
    out = out_ref[...]
    agg = (agg_ref[0] + agg_ref[1]) * dinv_ref[...]
    m = _leaky(agg + jnp.dot(out, cr_ref[...],
                             preferred_element_type=jnp.float32) + cb_ref[...])
    gi = jnp.dot(m, wih_ref[...], preferred_element_type=jnp.float32) + bih_ref[...]
    gh = jnp.dot(out, whh_ref[...], preferred_element_type=jnp.float32) + bhh_ref[...]
    r = jax.nn.sigmoid(gi[:, 0:DIM] + gh[:, 0:DIM])
    z = jax.nn.sigmoid(gi[:, DIM:2 * DIM] + gh[:, DIM:2 * DIM])
    n = jnp.tanh(gi[:, 2 * DIM:] + r * gh[:, 2 * DIM:])
    newo = (1.0 - z) * n + z * out
    newo_ref[...] = newo


def _stem_body(rows_ref, w1_ref, b1_ref, w2_ref, b2_ref, o_ref):
    t = _leaky(jnp.dot(rows_ref[...], w1_ref[...],
                       preferred_element_type=jnp.float32) + b1_ref[...])
    o_ref[...] = jnp.dot(t, w2_ref[...],
                         preferred_element_type=jnp.float32) + b2_ref[...]


def _s2s_body(out_ref, bt_ref, wih_ref, bih_ref, whh_ref, bhh_ref,
              w3_ref, b3_ref, sout_ref):
    outT = jnp.transpose(out_ref[...])
    seg = jax.lax.broadcasted_iota(jnp.int32, (B, N), 0) == bt_ref[...]
    segf = seg.astype(jnp.float32)
    q_star = jnp.zeros((B, 2 * DIM), jnp.float32)
    hL = jnp.zeros((B, DIM), jnp.float32)
    cL = jnp.zeros((B, DIM), jnp.float32)
    for _ in range(3):
        gates = (jnp.dot(q_star, wih_ref[...], preferred_element_type=jnp.float32)
                 + bih_ref[...]
                 + jnp.dot(hL, whh_ref[...], preferred_element_type=jnp.float32)
                 + bhh_ref[...])
        ii = gates[:, 0:DIM]
        ff = gates[:, DIM:2 * DIM]
        gg = gates[:, 2 * DIM:3 * DIM]
        oo = gates[:, 3 * DIM:]
        cL = jax.nn.sigmoid(ff) * cL + jax.nn.sigmoid(ii) * jnp.tanh(gg)
        hL = jax.nn.sigmoid(oo) * jnp.tanh(cL)
        qbT = lax.dot_general(jnp.transpose(hL), segf, (((1,), (0,)), ((), ())),
                              preferred_element_type=jnp.float32)
        eT = jnp.sum(outT * qbT, axis=0, keepdims=True)
        em = jnp.where(seg, eT, -jnp.inf)
        mx = jnp.max(em, axis=1, keepdims=True)
        mx = jnp.where(jnp.isfinite(mx), mx, 0.0)
        p = jnp.where(seg, jnp.exp(eT - mx), 0.0)
        s = jnp.sum(p, axis=1, keepdims=True)
        s = jnp.where(s == 0.0, 1.0, s)
        a = p / s
        rvec = lax.dot_general(a, outT, (((1,), (1,)), ((), ())),
                               preferred_element_type=jnp.float32)
        q_star = jnp.concatenate([hL, rvec], axis=1)
    sout_ref[...] = jnp.dot(q_star, w3_ref[...],
                            preferred_element_type=jnp.float32) + b3_ref[...]


def _sc_gather_body(epw, mld, ch, table_hbm, idx_hbm, out_hbm, idx_v, rows_v, sem):
    wid = lax.axis_index("c") * NSUB + lax.axis_index("s")
    nld = epw // mld
    spl = mld // ch
    pltpu.sync_copy(idx_hbm.at[wid], idx_v)

    def round_(g, c):
        def fire(k, c2):
            pltpu.async_copy(table_hbm.at[idx_v.at[g * spl + k]],
                             rows_v.at[pl.ds(k * ch, ch)], sem)
            return c2

        lax.fori_loop(0, spl, fire, 0)

        def drain(k, c2):
            pltpu.make_async_copy(table_hbm.at[idx_v.at[0]],
                                  rows_v.at[pl.ds(0, ch)], sem).wait()
            return c2

        lax.fori_loop(0, spl, drain, 0)
        pltpu.sync_copy(rows_v, out_hbm.at[pl.ds(wid * epw + g * mld, mld)])
        return c

    lax.fori_loop(0, nld, round_, 0)


def _sc_scatter_body(msg_hbm, idx_hbm, zero_hbm, agg_hbm, idx_v, msg_v, agg_sh, sem):
    cid = lax.axis_index("c")
    sid = lax.axis_index("s")
    wid = cid * NSUB + sid
    pltpu.sync_copy(idx_hbm.at[wid], idx_v)
    pltpu.sync_copy(zero_hbm, agg_sh.at[pl.ds(sid * NPT, NPT)])
    plsc.subcore_barrier()

    def body(g, c):
        pltpu.sync_copy(msg_hbm.at[pl.ds(wid * EPW + g * MLD, MLD)], msg_v)

        def sub(k, c2):
            pltpu.async_copy(msg_v.at[pl.ds(k * CH, CH)],
                             agg_sh.at[idx_v.at[g * SPL + k]], sem, add=True)
            return c2

        lax.fori_loop(0, SPL, sub, 0)

        def sdrain(k, c2):
            pltpu.make_async_copy(msg_v.at[pl.ds(0, CH)],
                                  agg_sh.at[idx_v.at[0]], sem).wait()
            return c2

        lax.fori_loop(0, SPL, sdrain, 0)
        return c

    lax.fori_loop(0, NLD, body, 0)
    plsc.subcore_barrier()
    pltpu.sync_copy(agg_sh.at[pl.ds(sid * NPT, NPT)],
                    agg_hbm.at[cid, pl.ds(sid * NPT, NPT)])


_MESH = plsc.VectorSubcoreMesh(core_axis_name="c", subcore_axis_name="s")


def _sc_gather(table, idx, rows_out_shape, epw, mld, ch):
    dt = table.dtype
    nch = epw // ch
    return pl.kernel(
        functools.partial(_sc_gather_body, epw, mld, ch),
        out_type=jax.ShapeDtypeStruct(rows_out_shape, dt),
        mesh=_MESH,
        compiler_params=pltpu.CompilerParams(use_tc_tiling_on_sc=False),
        scratch_types=[
            pltpu.VMEM((nch, ch), jnp.int32),
            pltpu.VMEM((mld, DIM), dt),
            pltpu.SemaphoreType.DMA,
        ],
    )(table, idx)


def _sc_scatter(msg, idx, zero):
    return pl.kernel(
        _sc_scatter_body,
        out_type=jax.ShapeDtypeStruct((NCORE, N, DIM), jnp.float32),
        mesh=_MESH,
        compiler_params=pltpu.CompilerParams(use_tc_tiling_on_sc=False),
        scratch_types=[
            pltpu.VMEM((NCH, CH), jnp.int32),
            pltpu.VMEM((MLD, DIM), jnp.float32),
            pltpu.VMEM_SHARED((N, DIM), jnp.float32),
            pltpu.SemaphoreType.DMA,
        ],
    )(msg, idx, zero)


def kernel(x, edge_index, edge_attr, batch, stems, stems_batch, slices_x, lin0_w, lin0_b, net_w1, net_b1, net_w2, net_b2, conv_root, conv_bias, gru_w_ih, gru_w_hh, gru_b_ih, gru_b_hh, lin1_w, lin1_b, lin2_w, lin2_b, s2s_w_ih, s2s_w_hh, s2s_b_ih, s2s_b_hh, lin3_w, lin3_b):
    src = edge_index[0]
    dst = edge_index[1]
    src_r = src.reshape(NW, NCH, CH)
    dst_r = dst.reshape(NW, NCH, CH)
    ea4 = edge_attr.reshape(E4, 16)
    eye4 = jnp.eye(4, dtype=jnp.float32)
    w1x = jnp.kron(eye4, net_w1)
    b1x = jnp.tile(net_b1, 4).reshape(1, 512)
    w2x = jnp.stack([jnp.kron(eye4, net_w2[:, 32 * i:32 * i + 32])
                     for i in range(DIM)]).astype(jnp.bfloat16)
    b2x = jnp.stack([jnp.tile(net_b2[32 * i:32 * i + 32], 4).reshape(1, 128)
                     for i in range(DIM)])
    lane = jnp.arange(128, dtype=jnp.int32)
    eis = jnp.stack([((lane[:, None] // 32 == lane[None, :] // 32)
                      & (lane[:, None] % 32 == i)).astype(jnp.bfloat16)
                     for i in range(DIM)])
    zero_t = jnp.zeros((NPT, DIM), jnp.float32)

    out = pl.pallas_call(
        _lin0_body,
        out_shape=jax.ShapeDtypeStruct((N, DIM), jnp.float32),
    )(x, lin0_w, lin0_b.reshape(1, DIM))

    w_edge = pl.pallas_call(
        _prep_w_body,
        grid=(E4 // RB,),
        in_specs=[
            pl.BlockSpec((RB, 16), lambda i: (i, 0)),
            pl.BlockSpec((16, 512), lambda i: (0, 0)),
            pl.BlockSpec((1, 512), lambda i: (0, 0)),
            pl.BlockSpec((DIM, 512, 128), lambda i: (0, 0, 0)),
            pl.BlockSpec((DIM, 1, 128), lambda i: (0, 0, 0)),
        ],
        out_specs=pl.BlockSpec((DIM, RB, 128), lambda i: (0, i, 0)),
        out_shape=jax.ShapeDtypeStruct((DIM, E4, 128), jnp.bfloat16),
    )(ea4, w1x, b1x, w2x, b2x)

    aggc = _sc_scatter(jnp.ones((E, DIM), jnp.float32), dst_r, zero_t)
    cnt = aggc[0, :, 0:1] + aggc[1, :, 0:1]
    dinv = jnp.broadcast_to(1.0 / jnp.maximum(cnt, 1.0), (N, DIM))

    gru_call = pl.pallas_call(
        _gru_body,
        grid=(5,),
        in_specs=[
            pl.BlockSpec((NCORE, N // 5, DIM), lambda i: (0, i, 0)),
            pl.BlockSpec((N // 5, DIM), lambda i: (i, 0)),
            pl.BlockSpec((N // 5, DIM), lambda i: (i, 0)),
            pl.BlockSpec((DIM, DIM), lambda i: (0, 0)),
            pl.BlockSpec((1, DIM), lambda i: (0, 0)),
            pl.BlockSpec((DIM, 3 * DIM), lambda i: (0, 0)),
            pl.BlockSpec((1, 3 * DIM), lambda i: (0, 0)),
            pl.BlockSpec((DIM, 3 * DIM), lambda i: (0, 0)),
            pl.BlockSpec((1, 3 * DIM), lambda i: (0, 0)),
        ],
        out_specs=pl.BlockSpec((N // 5, DIM), lambda i: (i, 0)),
        out_shape=jax.ShapeDtypeStruct((N, DIM), jnp.float32),
    )

    einsum_call = pl.pallas_call(
        _einsum_body,
        grid=(E4 // RB,),
        in_specs=[
            pl.BlockSpec((RB, 128), lambda i: (i, 0)),
            pl.BlockSpec((DIM, RB, 128), lambda i: (0, i, 0)),
            pl.BlockSpec((DIM, 128, 128), lambda i: (0, 0, 0)),
        ],
        out_specs=pl.BlockSpec((RB, 128), lambda i: (i, 0)),
        out_shape=jax.ShapeDtypeStruct((E4, 128), jnp.float32),
    )

    cr = conv_root
    cb = conv_bias.reshape(1, DIM)
    wih = gru_w_ih.T
    bih = gru_b_ih.reshape(1, 3 * DIM)
    whh = gru_w_hh.T
    bhh = gru_b_hh.reshape(1, 3 * DIM)

    for _ in range(STEPS):
        xj = _sc_gather(out, src_r, (E, DIM), EPW, MLD, CH)
        msg4 = einsum_call(xj.reshape(E4, 128), w_edge, eis)
        agg2 = _sc_scatter(msg4.reshape(E, DIM), dst_r, zero_t)
        out = gru_call(agg2, out, dinv, cr, cb, wih, bih, whh, bhh)

    stem_idx = slices_x[stems_batch] + stems
    sidx = jnp.zeros((S_PAD,), jnp.int32).at[0:500].set(stem_idx)
    srows = _sc_gather(out, sidx.reshape(NW, 1, S_EPW), (S_PAD, DIM), S_EPW, S_EPW, S_EPW)
    per_stem_pad = pl.pallas_call(
        _stem_body,
        out_shape=jax.ShapeDtypeStruct((S_PAD, 105), jnp.float32),
    )(srows, lin1_w, lin1_b.reshape(1, 8 * DIM), lin2_w, lin2_b.reshape(1, 105))
    per_stem_out = per_stem_pad[0:500]

    sout = pl.pallas_call(
        _s2s_body,
        out_shape=jax.ShapeDtypeStruct((B, 1), jnp.float32),
    )(out, batch.reshape(1, N), s2s_w_ih.T, s2s_b_ih.reshape(1, 4 * DIM),
      s2s_w_hh.T, s2s_b_hh.reshape(1, 4 * DIM), lin3_w, lin3_b.reshape(1, 1))

    return (sout, per_stem_out)
